# Initial kernel scaffold; baseline (speedup 1.0000x reference)
import jax
import jax.numpy as jnp
from jax.experimental import pallas as pl


def kernel(x, edge_index, W1l, b1, W1r, W2l, b2, W2r, Wlin, blin):
    raise NotImplementedError("write your pallas kernel here")


# speedup vs baseline: 6.2477x; 6.2477x over previous
import functools

import jax
import jax.numpy as jnp
from jax import lax
from jax.experimental import pallas as pl
from jax.experimental.pallas import tpu as pltpu
from jax.experimental.pallas import tpu_sc as plsc

N = 10000
E = 160000
D_IN = 256
D_HID = 256
D_EMB = 128

NC = 2
NS = 16
LANES = 16

NPAD = 10240
C = 125
RPS1 = E // C // NS
RPS2 = E // C // (NC * NS)
IB = 8
ROWS_OUT = NPAD // NS

BM = 1024
GRID = NPAD // BM


def _tc1_body(x_ref, wl_ref, wr_ref, y_ref, xr_ref):
    xb = x_ref[...]
    y = jnp.dot(xb, wl_ref[...], preferred_element_type=jnp.float32)
    y_ref[0] = y[:, :128]
    y_ref[1] = y[:, 128:]
    xr_ref[...] = jnp.dot(xb, wr_ref[...], preferred_element_type=jnp.float32)


def _tc1(xp, W1l, W1r):
    return pl.pallas_call(
        _tc1_body,
        grid=(GRID,),
        in_specs=[
            pl.BlockSpec((BM, D_IN), lambda i: (i, 0)),
            pl.BlockSpec((D_IN, D_HID), lambda i: (0, 0)),
            pl.BlockSpec((D_IN, D_HID), lambda i: (0, 0)),
        ],
        out_specs=[
            pl.BlockSpec((2, BM, 128), lambda i: (0, i, 0)),
            pl.BlockSpec((BM, D_HID), lambda i: (i, 0)),
        ],
        out_shape=[
            jax.ShapeDtypeStruct((2, NPAD, 128), jnp.float32),
            jax.ShapeDtypeStruct((NPAD, D_HID), jnp.float32),
        ],
    )(xp, W1l, W1r)


def _tc2_body(acc_ref, cnt_ref, xr_ref, b1_ref, w2l_ref, w2r_ref,
              y2_ref, hr_ref):
    cnt = jnp.maximum(cnt_ref[...], 1.0)
    agg = jnp.concatenate([acc_ref[0], acc_ref[1]], axis=1) / cnt
    h = jnp.maximum(agg + b1_ref[...] + xr_ref[...], 0.0)
    y2_ref[...] = jnp.dot(h, w2l_ref[...], preferred_element_type=jnp.float32)
    hr_ref[...] = jnp.dot(h, w2r_ref[...], preferred_element_type=jnp.float32)


def _tc2(acc1, cnt2, xr, b1, W2l, W2r):
    return pl.pallas_call(
        _tc2_body,
        grid=(GRID,),
        in_specs=[
            pl.BlockSpec((2, BM, 128), lambda i: (0, i, 0)),
            pl.BlockSpec((BM, 1), lambda i: (i, 0)),
            pl.BlockSpec((BM, D_HID), lambda i: (i, 0)),
            pl.BlockSpec((1, D_HID), lambda i: (0, 0)),
            pl.BlockSpec((D_HID, D_EMB), lambda i: (0, 0)),
            pl.BlockSpec((D_HID, D_EMB), lambda i: (0, 0)),
        ],
        out_specs=[
            pl.BlockSpec((BM, D_EMB), lambda i: (i, 0)),
            pl.BlockSpec((BM, D_EMB), lambda i: (i, 0)),
        ],
        out_shape=[
            jax.ShapeDtypeStruct((NPAD, D_EMB), jnp.float32),
            jax.ShapeDtypeStruct((NPAD, D_EMB), jnp.float32),
        ],
    )(acc1, cnt2, xr, b1, W2l, W2r)


def _tc3_body(acc_ref, cnt_ref, hr_ref, b2_ref, wlt_ref, blin_ref,
              emb_ref, out_ref):
    cnt = jnp.maximum(cnt_ref[...], 1.0)
    emb = ((acc_ref[0] + acc_ref[1]) / cnt + b2_ref[...] + hr_ref[...])
    emb_ref[...] = emb
    out_ref[...] = (jnp.sum(emb * wlt_ref[...], axis=1, keepdims=True)
                    + blin_ref[...])


def _tc3(acc2, cnt2, hr, b2, wlt, blin2):
    return pl.pallas_call(
        _tc3_body,
        grid=(GRID,),
        in_specs=[
            pl.BlockSpec((2, BM, D_EMB), lambda i: (0, i, 0)),
            pl.BlockSpec((BM, 1), lambda i: (i, 0)),
            pl.BlockSpec((BM, D_EMB), lambda i: (i, 0)),
            pl.BlockSpec((1, D_EMB), lambda i: (0, 0)),
            pl.BlockSpec((1, D_EMB), lambda i: (0, 0)),
            pl.BlockSpec((1, 1), lambda i: (0, 0)),
        ],
        out_specs=[
            pl.BlockSpec((BM, D_EMB), lambda i: (i, 0)),
            pl.BlockSpec((BM, 1), lambda i: (i, 0)),
        ],
        out_shape=[
            jax.ShapeDtypeStruct((NPAD, D_EMB), jnp.float32),
            jax.ShapeDtypeStruct((NPAD, 1), jnp.float32),
        ],
    )(acc2, cnt2, hr, b2, wlt, blin2)


def _zero_zbuf(zbuf, nrows, ncols):
    zv = jnp.zeros((LANES,), jnp.float32)
    for r in range(nrows):
        for g in range(ncols // LANES):
            zbuf[r, pl.ds(g * LANES, LANES)] = zv


def _zero_shared(zbuf, shared, row0):
    def zloop(i, carry):
        pltpu.sync_copy(zbuf, shared.at[pl.ds(row0 + i * LANES, LANES)])
        return carry
    lax.fori_loop(0, ROWS_OUT // LANES, zloop, 0)


def _make_seg1():
    mesh = plsc.VectorSubcoreMesh(
        core_axis_name="c", subcore_axis_name="s",
        num_cores=NC, num_subcores=NS)

    EPSP = 10240
    CH = EPSP // 2

    out_type = (jax.ShapeDtypeStruct((2, NPAD, 128), jnp.float32),
                jax.ShapeDtypeStruct((NPAD,), jnp.float32))
    scratch = [
        pltpu.VMEM((IB, C), jnp.int32),
        pltpu.VMEM((IB, C), jnp.int32),
        pltpu.VMEM((C, 128), jnp.float32),
        pltpu.VMEM((LANES, 128), jnp.float32),
        pltpu.VMEM((CH,), jnp.int32),
        pltpu.VMEM((NPAD,), jnp.float32),
        pltpu.VMEM((ROWS_OUT,), jnp.float32),
        pltpu.VMEM((ROWS_OUT,), jnp.float32),
        pltpu.VMEM_SHARED((NPAD, 128), jnp.float32),
        pltpu.VMEM_SHARED((NS, 1, NPAD), jnp.float32),
        pltpu.SemaphoreType.DMA,
    ]

    def body(tbl_hbm, src_hbm, dst_hbm, dstf_hbm, acc_out, cnt_out,
             sbuf, dbuf, rbuf, zbuf, cdbuf, cpriv, cbuf, cacc,
             accs, cstage, sem):
        cid = lax.axis_index("c")
        sid = lax.axis_index("s")
        row0 = sid * ROWS_OUT

        _zero_zbuf(zbuf, LANES, 128)
        _zero_shared(zbuf, accs, row0)

        plsc.subcore_barrier()

        def eblock(b, carry):
            pltpu.sync_copy(src_hbm.at[cid, sid, pl.ds(b * IB, IB)], sbuf)
            pltpu.sync_copy(dst_hbm.at[sid, pl.ds(b * IB, IB)], dbuf)

            def echunk(j, carry2):
                pltpu.async_copy(tbl_hbm.at[sbuf.at[j]], rbuf, sem).wait()
                pltpu.sync_copy(rbuf, accs.at[dbuf.at[j]], add=True)
                return carry2
            return lax.fori_loop(0, IB, echunk, carry)
        lax.fori_loop(0, RPS1 // IB, eblock, 0)

        @pl.when(cid == 0)
        def _counts():
            zv = jnp.zeros((LANES,), jnp.float32)
            ones = jnp.full((LANES,), 1.0, jnp.float32)

            def zpriv(i, carry):
                cpriv[pl.ds(i * LANES, LANES)] = zv
                return carry
            lax.fori_loop(0, NPAD // LANES, zpriv, 0)

            for half in range(2):
                pltpu.sync_copy(dstf_hbm.at[sid, 0, pl.ds(half * CH, CH)],
                                cdbuf)

                def cadd(i, carry):
                    idx = cdbuf[pl.ds(i * LANES, LANES)]
                    plsc.addupdate_scatter(cpriv, [idx], ones)
                    return carry
                lax.fori_loop(0, CH // LANES, cadd, 0)

            pltpu.sync_copy(cpriv, cstage.at[sid, 0])

        plsc.subcore_barrier()

        pltpu.sync_copy(accs.at[pl.ds(row0, ROWS_OUT)],
                        acc_out.at[cid, pl.ds(row0, ROWS_OUT)])

        @pl.when(cid == 0)
        def _cred():
            zv = jnp.zeros((LANES,), jnp.float32)

            def zacc(i, carry):
                cacc[pl.ds(i * LANES, LANES)] = zv
                return carry
            lax.fori_loop(0, ROWS_OUT // LANES, zacc, 0)

            for r in range(NS):
                pltpu.sync_copy(cstage.at[r, 0, pl.ds(row0, ROWS_OUT)], cbuf)

                def radd(i, carry):
                    s = pl.ds(i * LANES, LANES)
                    cacc[s] = cacc[s] + cbuf[s]
                    return carry
                lax.fori_loop(0, ROWS_OUT // LANES, radd, 0)

            pltpu.sync_copy(cacc, cnt_out.at[pl.ds(row0, ROWS_OUT)])

    return pl.kernel(body, out_type=out_type, mesh=mesh,
                     scratch_types=scratch,
                     compiler_params=pltpu.CompilerParams(
                         needs_layout_passes=False))


def _make_seg2():
    mesh = plsc.VectorSubcoreMesh(
        core_axis_name="c", subcore_axis_name="s",
        num_cores=NC, num_subcores=NS)

    out_type = (jax.ShapeDtypeStruct((2, NPAD, D_EMB), jnp.float32),)
    scratch = [
        pltpu.VMEM((IB, C), jnp.int32),
        pltpu.VMEM((IB, C), jnp.int32),
        pltpu.VMEM((C, D_EMB), jnp.float32),
        pltpu.VMEM((LANES, D_EMB), jnp.float32),
        pltpu.VMEM_SHARED((NPAD, D_EMB), jnp.float32),
        pltpu.SemaphoreType.DMA,
    ]

    def body(tbl_hbm, src_hbm, dst_hbm, acc_out,
             sbuf, dbuf, rbuf, zbuf, accs, sem):
        cid = lax.axis_index("c")
        sid = lax.axis_index("s")
        row0 = sid * ROWS_OUT

        _zero_zbuf(zbuf, LANES, D_EMB)
        _zero_shared(zbuf, accs, row0)

        plsc.subcore_barrier()

        def eblock(b, carry):
            pltpu.sync_copy(src_hbm.at[cid, sid, pl.ds(b * IB, IB)], sbuf)
            pltpu.sync_copy(dst_hbm.at[cid, sid, pl.ds(b * IB, IB)], dbuf)

            def echunk(j, carry2):
                pltpu.async_copy(tbl_hbm.at[sbuf.at[j]], rbuf, sem).wait()
                pltpu.sync_copy(rbuf, accs.at[dbuf.at[j]], add=True)
                return carry2
            return lax.fori_loop(0, IB, echunk, carry)
        lax.fori_loop(0, RPS2 // IB, eblock, 0)

        plsc.subcore_barrier()

        pltpu.sync_copy(accs.at[pl.ds(row0, ROWS_OUT)],
                        acc_out.at[cid, pl.ds(row0, ROWS_OUT)])

    return pl.kernel(body, out_type=out_type, mesh=mesh,
                     scratch_types=scratch,
                     compiler_params=pltpu.CompilerParams(
                         needs_layout_passes=False))


@functools.lru_cache(maxsize=None)
def _seg(layer):
    return _make_seg1() if layer == 1 else _make_seg2()


@jax.jit
def kernel(x, edge_index, W1l, b1, W1r, W2l, b2, W2r, Wlin, blin):
    xp = jnp.zeros((NPAD, D_IN), jnp.float32).at[:N].set(x)
    src = edge_index[0].reshape(NS, RPS1, C)
    dst = edge_index[1].reshape(NS, RPS1, C)
    src1 = jnp.stack([src, src + NPAD])
    src2 = edge_index[0].reshape(NC, NS, RPS2, C)
    dst2 = edge_index[1].reshape(NC, NS, RPS2, C)

    y1, xr = _tc1(xp, W1l, W1r)
    dstf = jnp.pad(edge_index[1].reshape(NS, E // NS),
                   ((0, 0), (0, 10240 - E // NS)),
                   constant_values=N).reshape(NS, 1, 10240)
    acc1, cnt = _seg(1)(y1.reshape(2 * NPAD, 128), src1, dst, dstf)
    cnt2 = cnt[:, None]

    y2, hr = _tc2(acc1, cnt2, xr, b1.reshape(1, D_HID), W2l, W2r)
    seg2_out = _seg(2)(y2, src2, dst2)
    acc2 = seg2_out[0] if isinstance(seg2_out, (list, tuple)) else seg2_out

    embp, outp = _tc3(acc2, cnt2, hr, b2.reshape(1, D_EMB),
                      Wlin.reshape(1, D_EMB), blin.reshape(1, 1))
    return embp[:N], outp[:N, 0]


# speedup vs baseline: 7.7626x; 1.2425x over previous
import functools

import jax
import jax.numpy as jnp
from jax import lax
from jax.experimental import pallas as pl
from jax.experimental.pallas import tpu as pltpu
from jax.experimental.pallas import tpu_sc as plsc

N = 10000
E = 160000
D_IN = 256
D_HID = 256
D_EMB = 128

NC = 2
NS = 16
LANES = 16

NPAD = 10240
C = 125
RPS1 = E // C // NS
RPS2 = E // C // (NC * NS)
IB = 8
ROWS_OUT = NPAD // NS

BM = 1024
GRID = NPAD // BM


def _tc1_body(x_ref, wl_ref, wr_ref, y_ref, xr_ref):
    xb = x_ref[...]
    y = jnp.dot(xb, wl_ref[...], preferred_element_type=jnp.float32)
    y_ref[0] = y[:, :128]
    y_ref[1] = y[:, 128:]
    xr_ref[...] = jnp.dot(xb, wr_ref[...], preferred_element_type=jnp.float32)


def _tc1(xp, W1l, W1r):
    return pl.pallas_call(
        _tc1_body,
        grid=(GRID,),
        in_specs=[
            pl.BlockSpec((BM, D_IN), lambda i: (i, 0)),
            pl.BlockSpec((D_IN, D_HID), lambda i: (0, 0)),
            pl.BlockSpec((D_IN, D_HID), lambda i: (0, 0)),
        ],
        out_specs=[
            pl.BlockSpec((2, BM, 128), lambda i: (0, i, 0)),
            pl.BlockSpec((BM, D_HID), lambda i: (i, 0)),
        ],
        out_shape=[
            jax.ShapeDtypeStruct((2, NPAD, 128), jnp.float32),
            jax.ShapeDtypeStruct((NPAD, D_HID), jnp.float32),
        ],
    )(xp, W1l, W1r)


def _tc2_body(acc_ref, cnta_ref, cntb_ref, xr_ref, b1_ref, w2l_ref, w2r_ref,
              y2_ref, hr_ref):
    cnt = jnp.maximum(cnta_ref[...] + cntb_ref[...], 1.0)
    agg = jnp.concatenate([acc_ref[0], acc_ref[1]], axis=1) / cnt
    h = jnp.maximum(agg + b1_ref[...] + xr_ref[...], 0.0)
    y2_ref[...] = jnp.dot(h, w2l_ref[...], preferred_element_type=jnp.float32)
    hr_ref[...] = jnp.dot(h, w2r_ref[...], preferred_element_type=jnp.float32)


def _tc2(acc1, cnta, cntb, xr, b1, W2l, W2r):
    return pl.pallas_call(
        _tc2_body,
        grid=(GRID,),
        in_specs=[
            pl.BlockSpec((2, BM, 128), lambda i: (0, i, 0)),
            pl.BlockSpec((BM, 1), lambda i: (i, 0)),
            pl.BlockSpec((BM, 1), lambda i: (i, 0)),
            pl.BlockSpec((BM, D_HID), lambda i: (i, 0)),
            pl.BlockSpec((1, D_HID), lambda i: (0, 0)),
            pl.BlockSpec((D_HID, D_EMB), lambda i: (0, 0)),
            pl.BlockSpec((D_HID, D_EMB), lambda i: (0, 0)),
        ],
        out_specs=[
            pl.BlockSpec((BM, D_EMB), lambda i: (i, 0)),
            pl.BlockSpec((BM, D_EMB), lambda i: (i, 0)),
        ],
        out_shape=[
            jax.ShapeDtypeStruct((NPAD, D_EMB), jnp.float32),
            jax.ShapeDtypeStruct((NPAD, D_EMB), jnp.float32),
        ],
    )(acc1, cnta, cntb, xr, b1, W2l, W2r)


def _tc3_body(acc_ref, cnta_ref, cntb_ref, hr_ref, b2_ref, wlt_ref, blin_ref,
              emb_ref, out_ref):
    cnt = jnp.maximum(cnta_ref[...] + cntb_ref[...], 1.0)
    emb = ((acc_ref[0] + acc_ref[1]) / cnt + b2_ref[...] + hr_ref[...])
    emb_ref[...] = emb
    out_ref[...] = (jnp.sum(emb * wlt_ref[...], axis=1, keepdims=True)
                    + blin_ref[...])


def _tc3(acc2, cnta, cntb, hr, b2, wlt, blin2):
    return pl.pallas_call(
        _tc3_body,
        grid=(GRID,),
        in_specs=[
            pl.BlockSpec((2, BM, D_EMB), lambda i: (0, i, 0)),
            pl.BlockSpec((BM, 1), lambda i: (i, 0)),
            pl.BlockSpec((BM, 1), lambda i: (i, 0)),
            pl.BlockSpec((BM, D_EMB), lambda i: (i, 0)),
            pl.BlockSpec((1, D_EMB), lambda i: (0, 0)),
            pl.BlockSpec((1, D_EMB), lambda i: (0, 0)),
            pl.BlockSpec((1, 1), lambda i: (0, 0)),
        ],
        out_specs=[
            pl.BlockSpec((BM, D_EMB), lambda i: (i, 0)),
            pl.BlockSpec((BM, 1), lambda i: (i, 0)),
        ],
        out_shape=[
            jax.ShapeDtypeStruct((NPAD, D_EMB), jnp.float32),
            jax.ShapeDtypeStruct((NPAD, 1), jnp.float32),
        ],
    )(acc2, cnta, cntb, hr, b2, wlt, blin2)


def _zero_zbuf(zbuf, nrows, ncols):
    zv = jnp.zeros((LANES,), jnp.float32)
    for r in range(nrows):
        for g in range(ncols // LANES):
            zbuf[r, pl.ds(g * LANES, LANES)] = zv


def _zero_shared(zbuf, shared, row0):
    def zloop(i, carry):
        pltpu.sync_copy(zbuf, shared.at[pl.ds(row0 + i * LANES, LANES)])
        return carry
    lax.fori_loop(0, ROWS_OUT // LANES, zloop, 0)


CH = 5120


def _make_cnt():
    mesh = plsc.VectorSubcoreMesh(
        core_axis_name="c", subcore_axis_name="s",
        num_cores=NC, num_subcores=NS)

    out_type = (jax.ShapeDtypeStruct((2 * NPAD,), jnp.float32),)
    scratch = [
        pltpu.VMEM((CH,), jnp.int32),
        pltpu.VMEM((NPAD,), jnp.float32),
        pltpu.VMEM((ROWS_OUT,), jnp.float32),
        pltpu.VMEM((ROWS_OUT,), jnp.float32),
        pltpu.VMEM_SHARED((NS, 1, NPAD), jnp.float32),
    ]

    def body(dstp_hbm, cnt_out, cdbuf, cpriv, cbuf, cacc, cstage):
        cid = lax.axis_index("c")
        sid = lax.axis_index("s")
        row0 = sid * ROWS_OUT
        zv = jnp.zeros((LANES,), jnp.float32)
        ones = jnp.full((LANES,), 1.0, jnp.float32)

        def zpriv(i, carry):
            cpriv[pl.ds(i * LANES, LANES)] = zv
            return carry
        lax.fori_loop(0, NPAD // LANES, zpriv, 0)

        pltpu.sync_copy(dstp_hbm.at[cid, sid, 0], cdbuf)

        def cadd(i, carry):
            idx = cdbuf[pl.ds(i * LANES, LANES)]
            plsc.addupdate_scatter(cpriv, [idx], ones)
            return carry
        lax.fori_loop(0, CH // LANES, cadd, 0)

        pltpu.sync_copy(cpriv, cstage.at[sid, 0])
        plsc.subcore_barrier()

        def zacc(i, carry):
            cacc[pl.ds(i * LANES, LANES)] = zv
            return carry
        lax.fori_loop(0, ROWS_OUT // LANES, zacc, 0)

        for r in range(NS):
            pltpu.sync_copy(cstage.at[r, 0, pl.ds(row0, ROWS_OUT)], cbuf)

            def radd(i, carry):
                s = pl.ds(i * LANES, LANES)
                cacc[s] = cacc[s] + cbuf[s]
                return carry
            lax.fori_loop(0, ROWS_OUT // LANES, radd, 0)

        pltpu.sync_copy(cacc, cnt_out.at[pl.ds(cid * NPAD + row0, ROWS_OUT)])

    return pl.kernel(body, out_type=out_type, mesh=mesh,
                     scratch_types=scratch,
                     compiler_params=pltpu.CompilerParams(
                         needs_layout_passes=False))


def _make_seg(layer):
    mesh = plsc.VectorSubcoreMesh(
        core_axis_name="c", subcore_axis_name="s",
        num_cores=NC, num_subcores=NS)

    rps = RPS1 if layer == 1 else RPS2
    out_type = (jax.ShapeDtypeStruct((2, NPAD, 128), jnp.float32),)
    scratch = [
        pltpu.VMEM((IB, C), jnp.int32),
        pltpu.VMEM((IB, C), jnp.int32),
        pltpu.VMEM((C, 128), jnp.float32),
        pltpu.VMEM((C, 128), jnp.float32),
        pltpu.VMEM((LANES, 128), jnp.float32),
        pltpu.VMEM_SHARED((NPAD, 128), jnp.float32),
        pltpu.SemaphoreType.DMA,
        pltpu.SemaphoreType.DMA,
    ]

    def body(tbl_hbm, src_hbm, dst_hbm, acc_out,
             sbuf, dbuf, rbuf0, rbuf1, zbuf, accs, sem0, sem1):
        cid = lax.axis_index("c")
        sid = lax.axis_index("s")
        row0 = sid * ROWS_OUT

        _zero_zbuf(zbuf, LANES, 128)
        _zero_shared(zbuf, accs, row0)

        plsc.subcore_barrier()

        bufs = (rbuf0, rbuf1)
        sems = (sem0, sem1)

        def eblock(b, carry):
            if layer == 1:
                pltpu.sync_copy(src_hbm.at[cid, sid, pl.ds(b * IB, IB)], sbuf)
                pltpu.sync_copy(dst_hbm.at[sid, pl.ds(b * IB, IB)], dbuf)
            else:
                pltpu.sync_copy(src_hbm.at[cid, sid, pl.ds(b * IB, IB)], sbuf)
                pltpu.sync_copy(dst_hbm.at[cid, sid, pl.ds(b * IB, IB)], dbuf)

            descs = {0: pltpu.async_copy(tbl_hbm.at[sbuf.at[0]],
                                         bufs[0], sems[0])}
            for j in range(IB):
                descs[j].wait()
                if j + 1 < IB:
                    descs[j + 1] = pltpu.async_copy(
                        tbl_hbm.at[sbuf.at[j + 1]],
                        bufs[(j + 1) % 2], sems[(j + 1) % 2])
                pltpu.sync_copy(bufs[j % 2], accs.at[dbuf.at[j]], add=True)
            return carry
        lax.fori_loop(0, rps // IB, eblock, 0)

        plsc.subcore_barrier()

        pltpu.sync_copy(accs.at[pl.ds(row0, ROWS_OUT)],
                        acc_out.at[cid, pl.ds(row0, ROWS_OUT)])

    return pl.kernel(body, out_type=out_type, mesh=mesh,
                     scratch_types=scratch,
                     compiler_params=pltpu.CompilerParams(
                         needs_layout_passes=False))


@functools.lru_cache(maxsize=None)
def _seg(layer):
    return _make_cnt() if layer == 0 else _make_seg(layer)


@jax.jit
def kernel(x, edge_index, W1l, b1, W1r, W2l, b2, W2r, Wlin, blin):
    xp = jnp.zeros((NPAD, D_IN), jnp.float32).at[:N].set(x)
    src = edge_index[0].reshape(NS, RPS1, C)
    dst = edge_index[1].reshape(NS, RPS1, C)
    src1 = jnp.stack([src, src + NPAD])
    src2 = edge_index[0].reshape(NC, NS, RPS2, C)
    dst2 = edge_index[1].reshape(NC, NS, RPS2, C)

    dstp = jnp.pad(edge_index[1].reshape(NS, E // NS),
                   ((0, 0), (0, 2 * CH - E // NS)),
                   constant_values=N)
    dstp = jnp.transpose(dstp.reshape(NS, 2, CH), (1, 0, 2))
    dstp = dstp.reshape(2, NS, 1, CH)

    cnt_out = _seg(0)(dstp)
    cnt1d = cnt_out[0] if isinstance(cnt_out, (list, tuple)) else cnt_out
    cnta = cnt1d[:NPAD, None]
    cntb = cnt1d[NPAD:, None]

    y1, xr = _tc1(xp, W1l, W1r)
    seg1_out = _seg(1)(y1.reshape(2 * NPAD, 128), src1, dst)
    acc1 = seg1_out[0] if isinstance(seg1_out, (list, tuple)) else seg1_out

    y2, hr = _tc2(acc1, cnta, cntb, xr, b1.reshape(1, D_HID), W2l, W2r)
    seg2_out = _seg(2)(y2, src2, dst2)
    acc2 = seg2_out[0] if isinstance(seg2_out, (list, tuple)) else seg2_out

    embp, outp = _tc3(acc2, cnta, cntb, hr, b2.reshape(1, D_EMB),
                      Wlin.reshape(1, D_EMB), blin.reshape(1, 1))
    return embp[:N], outp[:N, 0]


# speedup vs baseline: 8.0359x; 1.0352x over previous
import functools

import jax
import jax.numpy as jnp
from jax import lax
from jax.experimental import pallas as pl
from jax.experimental.pallas import tpu as pltpu
from jax.experimental.pallas import tpu_sc as plsc

N = 10000
E = 160000
D_IN = 256
D_HID = 256
D_EMB = 128

NC = 2
NS = 16
LANES = 16

NPAD = 10240
C = 125
RPS1 = E // C // NS
RPS2 = E // C // (NC * NS)
IB = 8
ROWS_OUT = NPAD // NS

BM = 1024
GRID = NPAD // BM


def _tc1_body(x_ref, wl_ref, wr_ref, y_ref, xr_ref):
    xb = x_ref[...].astype(jnp.bfloat16)
    y = jnp.dot(xb, wl_ref[...].astype(jnp.bfloat16),
                preferred_element_type=jnp.float32)
    y_ref[0] = y[:, :128]
    y_ref[1] = y[:, 128:]
    xr_ref[...] = jnp.dot(xb, wr_ref[...].astype(jnp.bfloat16),
                          preferred_element_type=jnp.float32)


def _tc1(xp, W1l, W1r):
    return pl.pallas_call(
        _tc1_body,
        grid=(GRID,),
        in_specs=[
            pl.BlockSpec((BM, D_IN), lambda i: (i, 0)),
            pl.BlockSpec((D_IN, D_HID), lambda i: (0, 0)),
            pl.BlockSpec((D_IN, D_HID), lambda i: (0, 0)),
        ],
        out_specs=[
            pl.BlockSpec((2, BM, 128), lambda i: (0, i, 0)),
            pl.BlockSpec((BM, D_HID), lambda i: (i, 0)),
        ],
        out_shape=[
            jax.ShapeDtypeStruct((2, NPAD, 128), jnp.float32),
            jax.ShapeDtypeStruct((NPAD, D_HID), jnp.float32),
        ],
    )(xp, W1l, W1r)


def _tc2_body(acc_ref, cnta_ref, cntb_ref, xr_ref, b1_ref, w2l_ref, w2r_ref,
              y2_ref, hr_ref):
    cnt = jnp.maximum(cnta_ref[...] + cntb_ref[...], 1.0)
    agg = jnp.concatenate([acc_ref[0], acc_ref[1]], axis=1) / cnt
    h = jnp.maximum(agg + b1_ref[...] + xr_ref[...], 0.0).astype(jnp.bfloat16)
    y2_ref[...] = jnp.dot(h, w2l_ref[...].astype(jnp.bfloat16),
                          preferred_element_type=jnp.float32)
    hr_ref[...] = jnp.dot(h, w2r_ref[...].astype(jnp.bfloat16),
                          preferred_element_type=jnp.float32)


def _tc2(acc1, cnta, cntb, xr, b1, W2l, W2r):
    return pl.pallas_call(
        _tc2_body,
        grid=(GRID,),
        in_specs=[
            pl.BlockSpec((2, BM, 128), lambda i: (0, i, 0)),
            pl.BlockSpec((BM, 1), lambda i: (i, 0)),
            pl.BlockSpec((BM, 1), lambda i: (i, 0)),
            pl.BlockSpec((BM, D_HID), lambda i: (i, 0)),
            pl.BlockSpec((1, D_HID), lambda i: (0, 0)),
            pl.BlockSpec((D_HID, D_EMB), lambda i: (0, 0)),
            pl.BlockSpec((D_HID, D_EMB), lambda i: (0, 0)),
        ],
        out_specs=[
            pl.BlockSpec((BM, D_EMB), lambda i: (i, 0)),
            pl.BlockSpec((BM, D_EMB), lambda i: (i, 0)),
        ],
        out_shape=[
            jax.ShapeDtypeStruct((NPAD, D_EMB), jnp.float32),
            jax.ShapeDtypeStruct((NPAD, D_EMB), jnp.float32),
        ],
    )(acc1, cnta, cntb, xr, b1, W2l, W2r)


def _tc3_body(acc_ref, cnta_ref, cntb_ref, hr_ref, b2_ref, wlt_ref, blin_ref,
              emb_ref, out_ref):
    cnt = jnp.maximum(cnta_ref[...] + cntb_ref[...], 1.0)
    emb = ((acc_ref[0] + acc_ref[1]) / cnt + b2_ref[...] + hr_ref[...])
    emb_ref[...] = emb
    out_ref[...] = (jnp.sum(emb * wlt_ref[...], axis=1, keepdims=True)
                    + blin_ref[...])


def _tc3(acc2, cnta, cntb, hr, b2, wlt, blin2):
    return pl.pallas_call(
        _tc3_body,
        grid=(GRID,),
        in_specs=[
            pl.BlockSpec((2, BM, D_EMB), lambda i: (0, i, 0)),
            pl.BlockSpec((BM, 1), lambda i: (i, 0)),
            pl.BlockSpec((BM, 1), lambda i: (i, 0)),
            pl.BlockSpec((BM, D_EMB), lambda i: (i, 0)),
            pl.BlockSpec((1, D_EMB), lambda i: (0, 0)),
            pl.BlockSpec((1, D_EMB), lambda i: (0, 0)),
            pl.BlockSpec((1, 1), lambda i: (0, 0)),
        ],
        out_specs=[
            pl.BlockSpec((BM, D_EMB), lambda i: (i, 0)),
            pl.BlockSpec((BM, 1), lambda i: (i, 0)),
        ],
        out_shape=[
            jax.ShapeDtypeStruct((NPAD, D_EMB), jnp.float32),
            jax.ShapeDtypeStruct((NPAD, 1), jnp.float32),
        ],
    )(acc2, cnta, cntb, hr, b2, wlt, blin2)


def _zero_zbuf(zbuf, nrows, ncols):
    zv = jnp.zeros((LANES,), jnp.float32)
    for r in range(nrows):
        for g in range(ncols // LANES):
            zbuf[r, pl.ds(g * LANES, LANES)] = zv


def _zero_shared(zbuf, shared, row0):
    def zloop(i, carry):
        pltpu.sync_copy(zbuf, shared.at[pl.ds(row0 + i * LANES, LANES)])
        return carry
    lax.fori_loop(0, ROWS_OUT // LANES, zloop, 0)


CH = 5120


def _make_cnt():
    mesh = plsc.VectorSubcoreMesh(
        core_axis_name="c", subcore_axis_name="s",
        num_cores=NC, num_subcores=NS)

    out_type = (jax.ShapeDtypeStruct((2 * NPAD,), jnp.float32),)
    scratch = [
        pltpu.VMEM((CH,), jnp.int32),
        pltpu.VMEM((NPAD,), jnp.float32),
        pltpu.VMEM((ROWS_OUT,), jnp.float32),
        pltpu.VMEM((ROWS_OUT,), jnp.float32),
        pltpu.VMEM_SHARED((NS, 1, NPAD), jnp.float32),
    ]

    def body(dstp_hbm, cnt_out, cdbuf, cpriv, cbuf, cacc, cstage):
        cid = lax.axis_index("c")
        sid = lax.axis_index("s")
        row0 = sid * ROWS_OUT
        zv = jnp.zeros((LANES,), jnp.float32)
        ones = jnp.full((LANES,), 1.0, jnp.float32)

        def zpriv(i, carry):
            cpriv[pl.ds(i * LANES, LANES)] = zv
            return carry
        lax.fori_loop(0, NPAD // LANES, zpriv, 0)

        pltpu.sync_copy(dstp_hbm.at[cid, sid, 0], cdbuf)

        def cadd(i, carry):
            idx = cdbuf[pl.ds(i * LANES, LANES)]
            plsc.addupdate_scatter(cpriv, [idx], ones)
            return carry
        lax.fori_loop(0, CH // LANES, cadd, 0)

        pltpu.sync_copy(cpriv, cstage.at[sid, 0])
        plsc.subcore_barrier()

        def zacc(i, carry):
            cacc[pl.ds(i * LANES, LANES)] = zv
            return carry
        lax.fori_loop(0, ROWS_OUT // LANES, zacc, 0)

        for r in range(NS):
            pltpu.sync_copy(cstage.at[r, 0, pl.ds(row0, ROWS_OUT)], cbuf)

            def radd(i, carry):
                s = pl.ds(i * LANES, LANES)
                cacc[s] = cacc[s] + cbuf[s]
                return carry
            lax.fori_loop(0, ROWS_OUT // LANES, radd, 0)

        pltpu.sync_copy(cacc, cnt_out.at[pl.ds(cid * NPAD + row0, ROWS_OUT)])

    return pl.kernel(body, out_type=out_type, mesh=mesh,
                     scratch_types=scratch,
                     compiler_params=pltpu.CompilerParams(
                         needs_layout_passes=False))


def _make_seg(layer):
    mesh = plsc.VectorSubcoreMesh(
        core_axis_name="c", subcore_axis_name="s",
        num_cores=NC, num_subcores=NS)

    rps = RPS1 if layer == 1 else RPS2
    HB = 40
    out_type = (jax.ShapeDtypeStruct((2, NPAD, 128), jnp.float32),)
    scratch = [
        pltpu.VMEM((HB, C), jnp.int32),
        pltpu.VMEM((HB, C), jnp.int32),
        pltpu.VMEM((C, 128), jnp.float32),
        pltpu.VMEM((C, 128), jnp.float32),
        pltpu.VMEM((2 * LANES, 128), jnp.float32),
        pltpu.VMEM_SHARED((NPAD, 128), jnp.float32),
        pltpu.SemaphoreType.DMA,
        pltpu.SemaphoreType.DMA,
        pltpu.SemaphoreType.DMA,
        pltpu.SemaphoreType.DMA,
    ]

    def body(tbl_hbm, src_hbm, dst_hbm, acc_out,
             sbuf, dbuf, rbuf0, rbuf1, zbuf, accs,
             gsem0, gsem1, ssem0, ssem1):
        cid = lax.axis_index("c")
        sid = lax.axis_index("s")
        row0 = sid * ROWS_OUT

        _zero_zbuf(zbuf, 2 * LANES, 128)

        def zloop(i, carry):
            pltpu.sync_copy(
                zbuf, accs.at[pl.ds(row0 + i * 2 * LANES, 2 * LANES)])
            return carry
        lax.fori_loop(0, ROWS_OUT // (2 * LANES), zloop, 0)

        plsc.subcore_barrier()

        bufs = (rbuf0, rbuf1)
        gsems = (gsem0, gsem1)
        ssems = (ssem0, ssem1)

        for h in range(rps // HB):
            pltpu.sync_copy(src_hbm.at[cid, sid, pl.ds(h * HB, HB)], sbuf)
            if layer == 1:
                pltpu.sync_copy(dst_hbm.at[sid, pl.ds(h * HB, HB)], dbuf)
            else:
                pltpu.sync_copy(dst_hbm.at[cid, sid, pl.ds(h * HB, HB)],
                                dbuf)

            def eblk(b, carry):
                base = b * IB
                gd = {}
                sd = {}
                gd[0] = pltpu.make_async_copy(
                    tbl_hbm.at[sbuf.at[base]], bufs[0], gsems[0])
                gd[0].start()
                for j in range(IB):
                    gd[j].wait()
                    if j + 1 < IB:
                        if j >= 1:
                            sd[j - 1].wait()
                        gd[j + 1] = pltpu.make_async_copy(
                            tbl_hbm.at[sbuf.at[base + j + 1]],
                            bufs[(j + 1) % 2], gsems[(j + 1) % 2])
                        gd[j + 1].start()
                    sd[j] = pltpu.make_async_copy(
                        bufs[j % 2], accs.at[dbuf.at[base + j]],
                        ssems[j % 2])
                    sd[j].start(add=True)
                sd[IB - 2].wait()
                sd[IB - 1].wait()
                return carry
            lax.fori_loop(0, HB // IB, eblk, 0)

        plsc.subcore_barrier()

        pltpu.sync_copy(accs.at[pl.ds(row0, ROWS_OUT)],
                        acc_out.at[cid, pl.ds(row0, ROWS_OUT)])

    return pl.kernel(body, out_type=out_type, mesh=mesh,
                     scratch_types=scratch,
                     compiler_params=pltpu.CompilerParams(
                         needs_layout_passes=False))


@functools.lru_cache(maxsize=None)
def _seg(layer):
    return _make_cnt() if layer == 0 else _make_seg(layer)


@jax.jit
def kernel(x, edge_index, W1l, b1, W1r, W2l, b2, W2r, Wlin, blin):
    xp = jnp.zeros((NPAD, D_IN), jnp.float32).at[:N].set(x)
    src = edge_index[0].reshape(NS, RPS1, C)
    dst = edge_index[1].reshape(NS, RPS1, C)
    src1 = jnp.stack([src, src + NPAD])
    src2 = edge_index[0].reshape(NC, NS, RPS2, C)
    dst2 = edge_index[1].reshape(NC, NS, RPS2, C)

    dstp = jnp.pad(edge_index[1].reshape(NS, E // NS),
                   ((0, 0), (0, 2 * CH - E // NS)),
                   constant_values=N)
    dstp = jnp.transpose(dstp.reshape(NS, 2, CH), (1, 0, 2))
    dstp = dstp.reshape(2, NS, 1, CH)

    cnt_out = _seg(0)(dstp)
    cnt1d = cnt_out[0] if isinstance(cnt_out, (list, tuple)) else cnt_out
    cnta = cnt1d[:NPAD, None]
    cntb = cnt1d[NPAD:, None]

    y1, xr = _tc1(xp, W1l, W1r)
    seg1_out = _seg(1)(y1.reshape(2 * NPAD, 128), src1, dst)
    acc1 = seg1_out[0] if isinstance(seg1_out, (list, tuple)) else seg1_out

    y2, hr = _tc2(acc1, cnta, cntb, xr, b1.reshape(1, D_HID), W2l, W2r)
    seg2_out = _seg(2)(y2, src2, dst2)
    acc2 = seg2_out[0] if isinstance(seg2_out, (list, tuple)) else seg2_out

    embp, outp = _tc3(acc2, cnta, cntb, hr, b2.reshape(1, D_EMB),
                      Wlin.reshape(1, D_EMB), blin.reshape(1, 1))
    return embp[:N], outp[:N, 0]


# speedup vs baseline: 8.0492x; 1.0016x over previous
import functools

import jax
import jax.numpy as jnp
from jax import lax
from jax.experimental import pallas as pl
from jax.experimental.pallas import tpu as pltpu
from jax.experimental.pallas import tpu_sc as plsc

N = 10000
E = 160000
D_IN = 256
D_HID = 256
D_EMB = 128

NC = 2
NS = 16
LANES = 16

NPAD = 10240
C = 125
RPS1 = E // C // NS
RPS2 = E // C // (NC * NS)
IB = 8
ROWS_OUT = NPAD // NS

BM = 1024
GRID = NPAD // BM


def _tc1_body(x_ref, wl_ref, wr_ref, y_ref, xr_ref):
    xb = x_ref[...].astype(jnp.bfloat16)
    y = jnp.dot(xb, wl_ref[...].astype(jnp.bfloat16),
                preferred_element_type=jnp.float32)
    y_ref[0] = y[:, :128]
    y_ref[1] = y[:, 128:]
    xr_ref[...] = jnp.dot(xb, wr_ref[...].astype(jnp.bfloat16),
                          preferred_element_type=jnp.float32)


def _tc1(xp, W1l, W1r):
    return pl.pallas_call(
        _tc1_body,
        grid=(GRID,),
        in_specs=[
            pl.BlockSpec((BM, D_IN), lambda i: (i, 0)),
            pl.BlockSpec((D_IN, D_HID), lambda i: (0, 0)),
            pl.BlockSpec((D_IN, D_HID), lambda i: (0, 0)),
        ],
        out_specs=[
            pl.BlockSpec((2, BM, 128), lambda i: (0, i, 0)),
            pl.BlockSpec((BM, D_HID), lambda i: (i, 0)),
        ],
        out_shape=[
            jax.ShapeDtypeStruct((2, NPAD, 128), jnp.float32),
            jax.ShapeDtypeStruct((NPAD, D_HID), jnp.float32),
        ],
    )(xp, W1l, W1r)


def _tc2_body(acc_ref, cnta_ref, cntb_ref, xr_ref, b1_ref, w2l_ref, w2r_ref,
              y2_ref, hr_ref):
    cnt = jnp.maximum(cnta_ref[...] + cntb_ref[...], 1.0)
    agg = jnp.concatenate([acc_ref[0], acc_ref[1]], axis=1) / cnt
    h = jnp.maximum(agg + b1_ref[...] + xr_ref[...], 0.0).astype(jnp.bfloat16)
    y2_ref[...] = jnp.dot(h, w2l_ref[...].astype(jnp.bfloat16),
                          preferred_element_type=jnp.float32)
    hr_ref[...] = jnp.dot(h, w2r_ref[...].astype(jnp.bfloat16),
                          preferred_element_type=jnp.float32)


def _tc2(acc1, cnta, cntb, xr, b1, W2l, W2r):
    return pl.pallas_call(
        _tc2_body,
        grid=(GRID,),
        in_specs=[
            pl.BlockSpec((2, BM, 128), lambda i: (0, i, 0)),
            pl.BlockSpec((BM, 1), lambda i: (i, 0)),
            pl.BlockSpec((BM, 1), lambda i: (i, 0)),
            pl.BlockSpec((BM, D_HID), lambda i: (i, 0)),
            pl.BlockSpec((1, D_HID), lambda i: (0, 0)),
            pl.BlockSpec((D_HID, D_EMB), lambda i: (0, 0)),
            pl.BlockSpec((D_HID, D_EMB), lambda i: (0, 0)),
        ],
        out_specs=[
            pl.BlockSpec((BM, D_EMB), lambda i: (i, 0)),
            pl.BlockSpec((BM, D_EMB), lambda i: (i, 0)),
        ],
        out_shape=[
            jax.ShapeDtypeStruct((NPAD, D_EMB), jnp.float32),
            jax.ShapeDtypeStruct((NPAD, D_EMB), jnp.float32),
        ],
    )(acc1, cnta, cntb, xr, b1, W2l, W2r)


def _tc3_body(acc_ref, cnta_ref, cntb_ref, hr_ref, b2_ref, wlt_ref, blin_ref,
              emb_ref, out_ref):
    cnt = jnp.maximum(cnta_ref[...] + cntb_ref[...], 1.0)
    emb = ((acc_ref[0].astype(jnp.float32) + acc_ref[1].astype(jnp.float32))
           / cnt + b2_ref[...] + hr_ref[...])
    emb_ref[...] = emb
    out_ref[...] = (jnp.sum(emb * wlt_ref[...], axis=1, keepdims=True)
                    + blin_ref[...])


def _tc3(acc2, cnta, cntb, hr, b2, wlt, blin2):
    return pl.pallas_call(
        _tc3_body,
        grid=(GRID,),
        in_specs=[
            pl.BlockSpec((2, BM, D_EMB), lambda i: (0, i, 0)),
            pl.BlockSpec((BM, 1), lambda i: (i, 0)),
            pl.BlockSpec((BM, 1), lambda i: (i, 0)),
            pl.BlockSpec((BM, D_EMB), lambda i: (i, 0)),
            pl.BlockSpec((1, D_EMB), lambda i: (0, 0)),
            pl.BlockSpec((1, D_EMB), lambda i: (0, 0)),
            pl.BlockSpec((1, 1), lambda i: (0, 0)),
        ],
        out_specs=[
            pl.BlockSpec((BM, D_EMB), lambda i: (i, 0)),
            pl.BlockSpec((BM, 1), lambda i: (i, 0)),
        ],
        out_shape=[
            jax.ShapeDtypeStruct((NPAD, D_EMB), jnp.float32),
            jax.ShapeDtypeStruct((NPAD, 1), jnp.float32),
        ],
    )(acc2, cnta, cntb, hr, b2, wlt, blin2)


def _zero_zbuf_f32(zbuf, nrows, ncols):
    zv = jnp.zeros((LANES,), jnp.float32)
    for r in range(nrows):
        for g in range(ncols // LANES):
            zbuf[r, pl.ds(g * LANES, LANES)] = zv


def _zero_zbuf_bf16_3d(zbuf, nrows, ncols):
    zv = jnp.zeros((2 * LANES,), jnp.bfloat16)
    for r in range(nrows):
        for s in range(2):
            for g in range(ncols // (2 * LANES)):
                zbuf[r, s, pl.ds(g * 2 * LANES, 2 * LANES)] = zv


def _zero_shared(zbuf, shared, row0):
    def zloop(i, carry):
        pltpu.sync_copy(zbuf, shared.at[pl.ds(row0 + i * LANES, LANES)])
        return carry
    lax.fori_loop(0, ROWS_OUT // LANES, zloop, 0)


CH = 5120


def _make_cnt():
    mesh = plsc.VectorSubcoreMesh(
        core_axis_name="c", subcore_axis_name="s",
        num_cores=NC, num_subcores=NS)

    out_type = (jax.ShapeDtypeStruct((2 * NPAD,), jnp.float32),)
    scratch = [
        pltpu.VMEM((CH,), jnp.int32),
        pltpu.VMEM((NPAD,), jnp.float32),
        pltpu.VMEM((ROWS_OUT,), jnp.float32),
        pltpu.VMEM((ROWS_OUT,), jnp.float32),
        pltpu.VMEM_SHARED((NS, 1, NPAD), jnp.float32),
    ]

    def body(dstp_hbm, cnt_out, cdbuf, cpriv, cbuf, cacc, cstage):
        cid = lax.axis_index("c")
        sid = lax.axis_index("s")
        row0 = sid * ROWS_OUT
        zv = jnp.zeros((LANES,), jnp.float32)
        ones = jnp.full((LANES,), 1.0, jnp.float32)

        def zpriv(i, carry):
            cpriv[pl.ds(i * LANES, LANES)] = zv
            return carry
        lax.fori_loop(0, NPAD // LANES, zpriv, 0)

        pltpu.sync_copy(dstp_hbm.at[cid, sid, 0], cdbuf)

        def cadd(i, carry):
            idx = cdbuf[pl.ds(i * LANES, LANES)]
            plsc.addupdate_scatter(cpriv, [idx], ones)
            return carry
        lax.fori_loop(0, CH // LANES, cadd, 0)

        pltpu.sync_copy(cpriv, cstage.at[sid, 0])
        plsc.subcore_barrier()

        def zacc(i, carry):
            cacc[pl.ds(i * LANES, LANES)] = zv
            return carry
        lax.fori_loop(0, ROWS_OUT // LANES, zacc, 0)

        for r in range(NS):
            pltpu.sync_copy(cstage.at[r, 0, pl.ds(row0, ROWS_OUT)], cbuf)

            def radd(i, carry):
                s = pl.ds(i * LANES, LANES)
                cacc[s] = cacc[s] + cbuf[s]
                return carry
            lax.fori_loop(0, ROWS_OUT // LANES, radd, 0)

        pltpu.sync_copy(cacc, cnt_out.at[pl.ds(cid * NPAD + row0, ROWS_OUT)])

    return pl.kernel(body, out_type=out_type, mesh=mesh,
                     scratch_types=scratch,
                     compiler_params=pltpu.CompilerParams(
                         needs_layout_passes=False))


def _make_seg(layer):
    mesh = plsc.VectorSubcoreMesh(
        core_axis_name="c", subcore_axis_name="s",
        num_cores=NC, num_subcores=NS)

    rps = RPS1 if layer == 1 else RPS2
    HB = 40
    out_type = (jax.ShapeDtypeStruct((2, NPAD, 128), jnp.float32),)
    scratch = [
        pltpu.VMEM((HB, C), jnp.int32),
        pltpu.VMEM((HB, C), jnp.int32),
        pltpu.VMEM((C, 128), jnp.float32),
        pltpu.VMEM((C, 128), jnp.float32),
        pltpu.VMEM((2 * LANES, 128), jnp.float32),
        pltpu.VMEM_SHARED((NPAD, 128), jnp.float32),
        pltpu.SemaphoreType.DMA,
        pltpu.SemaphoreType.DMA,
        pltpu.SemaphoreType.DMA,
        pltpu.SemaphoreType.DMA,
    ]

    def body(tbl_hbm, src_hbm, dst_hbm, acc_out,
             sbuf, dbuf, rbuf0, rbuf1, zbuf, accs,
             gsem0, gsem1, ssem0, ssem1):
        cid = lax.axis_index("c")
        sid = lax.axis_index("s")
        row0 = sid * ROWS_OUT

        _zero_zbuf_f32(zbuf, 2 * LANES, 128)

        def zloop(i, carry):
            pltpu.sync_copy(
                zbuf, accs.at[pl.ds(row0 + i * 2 * LANES, 2 * LANES)])
            return carry
        lax.fori_loop(0, ROWS_OUT // (2 * LANES), zloop, 0)

        plsc.subcore_barrier()

        bufs = (rbuf0, rbuf1)
        gsems = (gsem0, gsem1)
        ssems = (ssem0, ssem1)

        for h in range(rps // HB):
            pltpu.sync_copy(src_hbm.at[cid, sid, pl.ds(h * HB, HB)], sbuf)
            if layer == 1:
                pltpu.sync_copy(dst_hbm.at[sid, pl.ds(h * HB, HB)], dbuf)
            else:
                pltpu.sync_copy(dst_hbm.at[cid, sid, pl.ds(h * HB, HB)],
                                dbuf)

            def eblk(b, carry):
                base = b * IB
                gd = {}
                sd = {}
                gd[0] = pltpu.make_async_copy(
                    tbl_hbm.at[sbuf.at[base]], bufs[0], gsems[0])
                gd[0].start()
                for j in range(IB):
                    gd[j].wait()
                    if j + 1 < IB:
                        if j >= 1:
                            sd[j - 1].wait()
                        gd[j + 1] = pltpu.make_async_copy(
                            tbl_hbm.at[sbuf.at[base + j + 1]],
                            bufs[(j + 1) % 2], gsems[(j + 1) % 2])
                        gd[j + 1].start()
                    sd[j] = pltpu.make_async_copy(
                        bufs[j % 2], accs.at[dbuf.at[base + j]],
                        ssems[j % 2])
                    sd[j].start(add=True)
                sd[IB - 2].wait()
                sd[IB - 1].wait()
                return carry
            lax.fori_loop(0, HB // IB, eblk, 0)

        plsc.subcore_barrier()

        pltpu.sync_copy(accs.at[pl.ds(row0, ROWS_OUT)],
                        acc_out.at[cid, pl.ds(row0, ROWS_OUT)])

    return pl.kernel(body, out_type=out_type, mesh=mesh,
                     scratch_types=scratch,
                     compiler_params=pltpu.CompilerParams(
                         needs_layout_passes=False))


@functools.lru_cache(maxsize=None)
def _seg(layer):
    return _make_cnt() if layer == 0 else _make_seg(layer)


@jax.jit
def kernel(x, edge_index, W1l, b1, W1r, W2l, b2, W2r, Wlin, blin):
    xp = jnp.zeros((NPAD, D_IN), jnp.float32).at[:N].set(x)
    src = edge_index[0].reshape(NS, RPS1, C)
    dst = edge_index[1].reshape(NS, RPS1, C)
    src1 = jnp.stack([src, src + NPAD])
    src2 = edge_index[0].reshape(NC, NS, RPS2, C)
    dst2 = edge_index[1].reshape(NC, NS, RPS2, C)

    dstp = jnp.pad(edge_index[1].reshape(NS, E // NS),
                   ((0, 0), (0, 2 * CH - E // NS)),
                   constant_values=N)
    dstp = jnp.transpose(dstp.reshape(NS, 2, CH), (1, 0, 2))
    dstp = dstp.reshape(2, NS, 1, CH)

    cnt_out = _seg(0)(dstp)
    cnt1d = cnt_out[0] if isinstance(cnt_out, (list, tuple)) else cnt_out
    cnta = cnt1d[:NPAD, None]
    cntb = cnt1d[NPAD:, None]

    y1, xr = _tc1(xp, W1l, W1r)
    seg1_out = _seg(1)(y1.reshape(2 * NPAD, 128), src1, dst)
    acc1 = seg1_out[0] if isinstance(seg1_out, (list, tuple)) else seg1_out

    y2, hr = _tc2(acc1, cnta, cntb, xr, b1.reshape(1, D_HID), W2l, W2r)
    seg2_out = _seg(2)(y2, src2, dst2)
    acc2 = seg2_out[0] if isinstance(seg2_out, (list, tuple)) else seg2_out

    embp, outp = _tc3(acc2, cnta, cntb, hr, b2.reshape(1, D_EMB),
                      Wlin.reshape(1, D_EMB), blin.reshape(1, 1))
    return embp[:N], outp[:N, 0]


# speedup vs baseline: 8.2692x; 1.0273x over previous
import functools

import jax
import jax.numpy as jnp
from jax import lax
from jax.experimental import pallas as pl
from jax.experimental.pallas import tpu as pltpu
from jax.experimental.pallas import tpu_sc as plsc

N = 10000
E = 160000
D_IN = 256
D_HID = 256
D_EMB = 128

NC = 2
NS = 16
LANES = 16

NPAD = 10240
C = 125
RPS1 = E // C // NS
RPS2 = E // C // (NC * NS)
IB = 10
ROWS_OUT = NPAD // NS

BM = 2048
GRID = NPAD // BM


def _tc1_body(x_ref, wl_ref, wr_ref, y_ref, xr_ref):
    xb = x_ref[...].astype(jnp.bfloat16)
    y = jnp.dot(xb, wl_ref[...].astype(jnp.bfloat16),
                preferred_element_type=jnp.float32)
    y_ref[0] = y[:, :128]
    y_ref[1] = y[:, 128:]
    xr_ref[...] = jnp.dot(xb, wr_ref[...].astype(jnp.bfloat16),
                          preferred_element_type=jnp.float32)


def _tc1(xp, W1l, W1r):
    return pl.pallas_call(
        _tc1_body,
        grid=(GRID,),
        in_specs=[
            pl.BlockSpec((BM, D_IN), lambda i: (i, 0)),
            pl.BlockSpec((D_IN, D_HID), lambda i: (0, 0)),
            pl.BlockSpec((D_IN, D_HID), lambda i: (0, 0)),
        ],
        out_specs=[
            pl.BlockSpec((2, BM, 128), lambda i: (0, i, 0)),
            pl.BlockSpec((BM, D_HID), lambda i: (i, 0)),
        ],
        out_shape=[
            jax.ShapeDtypeStruct((2, NPAD, 128), jnp.float32),
            jax.ShapeDtypeStruct((NPAD, D_HID), jnp.float32),
        ],
    )(xp, W1l, W1r)


def _tc2_body(acc_ref, cnta_ref, cntb_ref, xr_ref, b1_ref, w2l_ref, w2r_ref,
              y2_ref, hr_ref):
    cnt = jnp.maximum(cnta_ref[...] + cntb_ref[...], 1.0)
    agg = jnp.concatenate([acc_ref[0], acc_ref[1]], axis=1) / cnt
    h = jnp.maximum(agg + b1_ref[...] + xr_ref[...], 0.0).astype(jnp.bfloat16)
    y2_ref[...] = jnp.dot(h, w2l_ref[...].astype(jnp.bfloat16),
                          preferred_element_type=jnp.float32)
    hr_ref[...] = jnp.dot(h, w2r_ref[...].astype(jnp.bfloat16),
                          preferred_element_type=jnp.float32)


def _tc2(acc1, cnta, cntb, xr, b1, W2l, W2r):
    return pl.pallas_call(
        _tc2_body,
        grid=(GRID,),
        in_specs=[
            pl.BlockSpec((2, BM, 128), lambda i: (0, i, 0)),
            pl.BlockSpec((BM, 1), lambda i: (i, 0)),
            pl.BlockSpec((BM, 1), lambda i: (i, 0)),
            pl.BlockSpec((BM, D_HID), lambda i: (i, 0)),
            pl.BlockSpec((1, D_HID), lambda i: (0, 0)),
            pl.BlockSpec((D_HID, D_EMB), lambda i: (0, 0)),
            pl.BlockSpec((D_HID, D_EMB), lambda i: (0, 0)),
        ],
        out_specs=[
            pl.BlockSpec((BM, D_EMB), lambda i: (i, 0)),
            pl.BlockSpec((BM, D_EMB), lambda i: (i, 0)),
        ],
        out_shape=[
            jax.ShapeDtypeStruct((NPAD, D_EMB), jnp.float32),
            jax.ShapeDtypeStruct((NPAD, D_EMB), jnp.float32),
        ],
    )(acc1, cnta, cntb, xr, b1, W2l, W2r)


def _tc3_body(acc_ref, cnta_ref, cntb_ref, hr_ref, b2_ref, wlt_ref, blin_ref,
              emb_ref, out_ref):
    cnt = jnp.maximum(cnta_ref[...] + cntb_ref[...], 1.0)
    emb = ((acc_ref[0].astype(jnp.float32) + acc_ref[1].astype(jnp.float32))
           / cnt + b2_ref[...] + hr_ref[...])
    emb_ref[...] = emb
    out_ref[...] = (jnp.sum(emb * wlt_ref[...], axis=1, keepdims=True)
                    + blin_ref[...])


def _tc3(acc2, cnta, cntb, hr, b2, wlt, blin2):
    return pl.pallas_call(
        _tc3_body,
        grid=(GRID,),
        in_specs=[
            pl.BlockSpec((2, BM, D_EMB), lambda i: (0, i, 0)),
            pl.BlockSpec((BM, 1), lambda i: (i, 0)),
            pl.BlockSpec((BM, 1), lambda i: (i, 0)),
            pl.BlockSpec((BM, D_EMB), lambda i: (i, 0)),
            pl.BlockSpec((1, D_EMB), lambda i: (0, 0)),
            pl.BlockSpec((1, D_EMB), lambda i: (0, 0)),
            pl.BlockSpec((1, 1), lambda i: (0, 0)),
        ],
        out_specs=[
            pl.BlockSpec((BM, D_EMB), lambda i: (i, 0)),
            pl.BlockSpec((BM, 1), lambda i: (i, 0)),
        ],
        out_shape=[
            jax.ShapeDtypeStruct((NPAD, D_EMB), jnp.float32),
            jax.ShapeDtypeStruct((NPAD, 1), jnp.float32),
        ],
    )(acc2, cnta, cntb, hr, b2, wlt, blin2)


def _zero_zbuf_f32(zbuf, nrows, ncols):
    zv = jnp.zeros((LANES,), jnp.float32)
    for r in range(nrows):
        for g in range(ncols // LANES):
            zbuf[r, pl.ds(g * LANES, LANES)] = zv


def _zero_zbuf_bf16_3d(zbuf, nrows, ncols):
    zv = jnp.zeros((2 * LANES,), jnp.bfloat16)
    for r in range(nrows):
        for s in range(2):
            for g in range(ncols // (2 * LANES)):
                zbuf[r, s, pl.ds(g * 2 * LANES, 2 * LANES)] = zv


def _zero_shared(zbuf, shared, row0):
    def zloop(i, carry):
        pltpu.sync_copy(zbuf, shared.at[pl.ds(row0 + i * LANES, LANES)])
        return carry
    lax.fori_loop(0, ROWS_OUT // LANES, zloop, 0)


CH = 5120


def _make_cnt():
    mesh = plsc.VectorSubcoreMesh(
        core_axis_name="c", subcore_axis_name="s",
        num_cores=NC, num_subcores=NS)

    out_type = (jax.ShapeDtypeStruct((2 * NPAD,), jnp.float32),)
    scratch = [
        pltpu.VMEM((CH,), jnp.int32),
        pltpu.VMEM((NPAD,), jnp.float32),
        pltpu.VMEM((ROWS_OUT,), jnp.float32),
        pltpu.VMEM((ROWS_OUT,), jnp.float32),
        pltpu.VMEM_SHARED((NS, 1, NPAD), jnp.float32),
    ]

    def body(dstp_hbm, cnt_out, cdbuf, cpriv, cbuf, cacc, cstage):
        cid = lax.axis_index("c")
        sid = lax.axis_index("s")
        row0 = sid * ROWS_OUT
        zv = jnp.zeros((LANES,), jnp.float32)
        ones = jnp.full((LANES,), 1.0, jnp.float32)

        def zpriv(i, carry):
            cpriv[pl.ds(i * LANES, LANES)] = zv
            return carry
        lax.fori_loop(0, NPAD // LANES, zpriv, 0)

        pltpu.sync_copy(dstp_hbm.at[cid, sid, 0], cdbuf)

        def cadd(i, carry):
            idx = cdbuf[pl.ds(i * LANES, LANES)]
            plsc.addupdate_scatter(cpriv, [idx], ones)
            return carry
        lax.fori_loop(0, CH // LANES, cadd, 0)

        pltpu.sync_copy(cpriv, cstage.at[sid, 0])
        plsc.subcore_barrier()

        def zacc(i, carry):
            cacc[pl.ds(i * LANES, LANES)] = zv
            return carry
        lax.fori_loop(0, ROWS_OUT // LANES, zacc, 0)

        for r in range(NS):
            pltpu.sync_copy(cstage.at[r, 0, pl.ds(row0, ROWS_OUT)], cbuf)

            def radd(i, carry):
                s = pl.ds(i * LANES, LANES)
                cacc[s] = cacc[s] + cbuf[s]
                return carry
            lax.fori_loop(0, ROWS_OUT // LANES, radd, 0)

        pltpu.sync_copy(cacc, cnt_out.at[pl.ds(cid * NPAD + row0, ROWS_OUT)])

    return pl.kernel(body, out_type=out_type, mesh=mesh,
                     scratch_types=scratch,
                     compiler_params=pltpu.CompilerParams(
                         needs_layout_passes=False))


def _make_seg(layer):
    mesh = plsc.VectorSubcoreMesh(
        core_axis_name="c", subcore_axis_name="s",
        num_cores=NC, num_subcores=NS)

    rps = RPS1 if layer == 1 else RPS2
    HB = 40
    out_type = (jax.ShapeDtypeStruct((2, NPAD, 128), jnp.float32),)
    scratch = [
        pltpu.VMEM((HB, C), jnp.int32),
        pltpu.VMEM((HB, C), jnp.int32),
        pltpu.VMEM((C, 128), jnp.float32),
        pltpu.VMEM((C, 128), jnp.float32),
        pltpu.VMEM((2 * LANES, 128), jnp.float32),
        pltpu.VMEM_SHARED((NPAD, 128), jnp.float32),
        pltpu.SemaphoreType.DMA,
        pltpu.SemaphoreType.DMA,
        pltpu.SemaphoreType.DMA,
        pltpu.SemaphoreType.DMA,
    ]

    def body(tbl_hbm, src_hbm, dst_hbm, acc_out,
             sbuf, dbuf, rbuf0, rbuf1, zbuf, accs,
             gsem0, gsem1, ssem0, ssem1):
        cid = lax.axis_index("c")
        sid = lax.axis_index("s")
        row0 = sid * ROWS_OUT

        _zero_zbuf_f32(zbuf, 2 * LANES, 128)

        def zloop(i, carry):
            pltpu.sync_copy(
                zbuf, accs.at[pl.ds(row0 + i * 2 * LANES, 2 * LANES)])
            return carry
        lax.fori_loop(0, ROWS_OUT // (2 * LANES), zloop, 0)

        plsc.subcore_barrier()

        bufs = (rbuf0, rbuf1)
        gsems = (gsem0, gsem1)
        ssems = (ssem0, ssem1)

        for h in range(rps // HB):
            pltpu.sync_copy(src_hbm.at[cid, sid, pl.ds(h * HB, HB)], sbuf)
            if layer == 1:
                pltpu.sync_copy(dst_hbm.at[sid, pl.ds(h * HB, HB)], dbuf)
            else:
                pltpu.sync_copy(dst_hbm.at[cid, sid, pl.ds(h * HB, HB)],
                                dbuf)

            def eblk(b, carry):
                base = b * IB
                gd = {}
                sd = {}
                gd[0] = pltpu.make_async_copy(
                    tbl_hbm.at[sbuf.at[base]], bufs[0], gsems[0])
                gd[0].start()
                for j in range(IB):
                    gd[j].wait()
                    if j + 1 < IB:
                        if j >= 1:
                            sd[j - 1].wait()
                        gd[j + 1] = pltpu.make_async_copy(
                            tbl_hbm.at[sbuf.at[base + j + 1]],
                            bufs[(j + 1) % 2], gsems[(j + 1) % 2])
                        gd[j + 1].start()
                    sd[j] = pltpu.make_async_copy(
                        bufs[j % 2], accs.at[dbuf.at[base + j]],
                        ssems[j % 2])
                    sd[j].start(add=True)
                sd[IB - 2].wait()
                sd[IB - 1].wait()
                return carry
            lax.fori_loop(0, HB // IB, eblk, 0)

        plsc.subcore_barrier()

        pltpu.sync_copy(accs.at[pl.ds(row0, ROWS_OUT)],
                        acc_out.at[cid, pl.ds(row0, ROWS_OUT)])

    return pl.kernel(body, out_type=out_type, mesh=mesh,
                     scratch_types=scratch,
                     compiler_params=pltpu.CompilerParams(
                         needs_layout_passes=False))


@functools.lru_cache(maxsize=None)
def _seg(layer):
    return _make_cnt() if layer == 0 else _make_seg(layer)


@jax.jit
def kernel(x, edge_index, W1l, b1, W1r, W2l, b2, W2r, Wlin, blin):
    xp = jnp.zeros((NPAD, D_IN), jnp.float32).at[:N].set(x)
    src = edge_index[0].reshape(NS, RPS1, C)
    dst = edge_index[1].reshape(NS, RPS1, C)
    src1 = jnp.stack([src, src + NPAD])
    src2 = edge_index[0].reshape(NC, NS, RPS2, C)
    dst2 = edge_index[1].reshape(NC, NS, RPS2, C)

    dstp = jnp.pad(edge_index[1].reshape(NS, E // NS),
                   ((0, 0), (0, 2 * CH - E // NS)),
                   constant_values=N)
    dstp = jnp.transpose(dstp.reshape(NS, 2, CH), (1, 0, 2))
    dstp = dstp.reshape(2, NS, 1, CH)

    cnt_out = _seg(0)(dstp)
    cnt1d = cnt_out[0] if isinstance(cnt_out, (list, tuple)) else cnt_out
    cnta = cnt1d[:NPAD, None]
    cntb = cnt1d[NPAD:, None]

    y1, xr = _tc1(xp, W1l, W1r)
    seg1_out = _seg(1)(y1.reshape(2 * NPAD, 128), src1, dst)
    acc1 = seg1_out[0] if isinstance(seg1_out, (list, tuple)) else seg1_out

    y2, hr = _tc2(acc1, cnta, cntb, xr, b1.reshape(1, D_HID), W2l, W2r)
    seg2_out = _seg(2)(y2, src2, dst2)
    acc2 = seg2_out[0] if isinstance(seg2_out, (list, tuple)) else seg2_out

    embp, outp = _tc3(acc2, cnta, cntb, hr, b2.reshape(1, D_EMB),
                      Wlin.reshape(1, D_EMB), blin.reshape(1, 1))
    return embp[:N], outp[:N, 0]
